# baseline (device time: 15612 ns/iter reference)
import jax
import jax.numpy as jnp
from jax import lax
from jax.experimental import pallas as pl
from jax.experimental.pallas import tpu as pltpu

N_DEV = 4
NS = 4
S1 = 0.7
S2 = 1.05

A_OWN, A_MATE, B_OWN, B_MATE, A_P2, B_P2 = range(6)


def kernel(x, dy):
    k_per, d_model = x.shape
    _, d_ff = dy.shape
    m_out = d_model // N_DEV
    blk = d_ff // (2 * NS)

    def body(x_ref, dy_ref, out_ref, xt_ref, dyb_ref, p_ref, q_send,
             recv_buf, send_sems, recv_sems):
        my = lax.axis_index("i")
        q1 = my ^ 1
        q2 = 3 - my

        barrier_sem = pltpu.get_barrier_semaphore()
        for peer in (q1, q2):
            pl.semaphore_signal(
                barrier_sem, inc=1,
                device_id=(peer,), device_id_type=pl.DeviceIdType.MESH,
            )

        xt_ref[:, :] = x_ref[:, :].T.astype(jnp.bfloat16)

        def cast_block(b):
            dyb_ref[:, pl.ds(b * blk, blk)] = dy_ref[
                :, pl.ds(b * blk, blk)
            ].astype(jnp.bfloat16)

        def chunk_gemm(c, b):
            return lax.dot_general(
                xt_ref[pl.ds(c * m_out, m_out), :],
                dyb_ref[:, pl.ds(b * blk, blk)],
                dimension_numbers=(((1,), (0,)), ((), ())),
                preferred_element_type=jnp.float32,
            )

        def quant(val, scale):
            return jnp.clip(
                jnp.round(val * (1.0 / scale)), -127.0, 127.0
            ).astype(jnp.int8)

        def store_local(c, b):
            p_ref[pl.ds(c * m_out, m_out), pl.ds(b * blk, blk)] = (
                chunk_gemm(c, b).astype(jnp.bfloat16)
            )

        def local(c, b):
            return p_ref[
                pl.ds(c * m_out, m_out), pl.ds(b * blk, blk)
            ].astype(jnp.float32)

        def make(slot, dest):
            return pltpu.make_async_remote_copy(
                src_ref=q_send.at[slot],
                dst_ref=recv_buf.at[slot],
                send_sem=send_sems.at[slot],
                recv_sem=recv_sems.at[slot],
                device_id=(dest,),
                device_id_type=pl.DeviceIdType.MESH,
            )

        def p1_send(c, b, slot, dest):
            q_send[slot, :, :] = quant(chunk_gemm(c, b), S1)
            s = make(slot, dest)
            s.start()
            return s

        sa_own, sa_mate, sb_own, sb_mate = [], [], [], []
        cast_block(0)
        q_send[A_OWN, :, :] = quant(chunk_gemm(q1, 0), S1)
        pl.semaphore_wait(barrier_sem, 2)
        s = make(A_OWN, q1)
        s.start()
        sa_own.append(s)
        for t in range(NS):
            base = 6 * t
            if t > 0:
                cast_block(t)
                sa_own.append(p1_send(q1, t, base + A_OWN, q1))
            cast_block(NS + t)
            sb_own.append(p1_send(q2, NS + t, base + B_OWN, q2))
            sa_mate.append(p1_send(3 - q1, t, base + A_MATE, q1))
            sb_mate.append(p1_send(q2 ^ 1, NS + t, base + B_MATE, q2))

        for t in range(NS):
            store_local(3 - my, t)
            store_local(my ^ 1, NS + t)
        for t in range(NS):
            store_local(my, t)
            store_local(my, NS + t)

        sa_p2, sb_p2 = [], []
        for t in range(NS):
            base = 6 * t
            sa_own[t].wait_recv()
            sa_mate[t].wait_recv()
            q_send[base + A_P2, :, :] = quant(
                local(3 - my, t)
                + recv_buf[base + A_MATE, :, :].astype(jnp.float32) * S1,
                S2,
            )
            s = make(base + A_P2, q2)
            s.start()
            sa_p2.append(s)
            sb_own[t].wait_recv()
            sb_mate[t].wait_recv()
            q_send[base + B_P2, :, :] = quant(
                local(my ^ 1, NS + t)
                + recv_buf[base + B_MATE, :, :].astype(jnp.float32) * S1,
                S2,
            )
            s = make(base + B_P2, q1)
            s.start()
            sb_p2.append(s)

        for t in range(NS):
            base = 6 * t
            sa_p2[t].wait_recv()
            out_ref[:, pl.ds(t * blk, blk)] = (
                local(my, t)
                + recv_buf[base + A_OWN, :, :].astype(jnp.float32) * S1
                + recv_buf[base + A_P2, :, :].astype(jnp.float32) * S2
            )
            sb_p2[t].wait_recv()
            out_ref[:, pl.ds((NS + t) * blk, blk)] = (
                local(my, NS + t)
                + recv_buf[base + B_OWN, :, :].astype(jnp.float32) * S1
                + recv_buf[base + B_P2, :, :].astype(jnp.float32) * S2
            )

        for group in (sa_own, sa_mate, sb_own, sb_mate, sa_p2, sb_p2):
            for s in group:
                s.wait_send()

    return pl.pallas_call(
        body,
        out_shape=jax.ShapeDtypeStruct((m_out, d_ff), jnp.float32),
        in_specs=[
            pl.BlockSpec(memory_space=pltpu.VMEM),
            pl.BlockSpec(memory_space=pltpu.VMEM),
        ],
        out_specs=pl.BlockSpec(memory_space=pltpu.VMEM),
        scratch_shapes=[
            pltpu.VMEM((d_model, k_per), jnp.bfloat16),
            pltpu.VMEM((k_per, d_ff), jnp.bfloat16),
            pltpu.VMEM((d_model, d_ff), jnp.bfloat16),
            pltpu.VMEM((6 * NS, m_out, blk), jnp.int8),
            pltpu.VMEM((6 * NS, m_out, blk), jnp.int8),
            pltpu.SemaphoreType.DMA((6 * NS,)),
            pltpu.SemaphoreType.DMA((6 * NS,)),
        ],
        compiler_params=pltpu.CompilerParams(collective_id=0),
    )(x, dy)


# device time: 14822 ns/iter; 1.0533x vs baseline; 1.0533x over previous
import jax
import jax.numpy as jnp
from jax import lax
from jax.experimental import pallas as pl
from jax.experimental.pallas import tpu as pltpu

N_DEV = 4
NS = 2
S1 = 0.7
S2 = 1.05

A_OWN, A_MATE, B_OWN, B_MATE, A_P2, B_P2 = range(6)


def kernel(x, dy):
    k_per, d_model = x.shape
    _, d_ff = dy.shape
    m_out = d_model // N_DEV
    blk = d_ff // (2 * NS)

    def body(x_ref, dy_ref, out_ref, xt_ref, dyb_ref, p_ref, q_send,
             recv_buf, send_sems, recv_sems):
        my = lax.axis_index("i")
        q1 = my ^ 1
        q2 = 3 - my

        barrier_sem = pltpu.get_barrier_semaphore()
        for peer in (q1, q2):
            pl.semaphore_signal(
                barrier_sem, inc=1,
                device_id=(peer,), device_id_type=pl.DeviceIdType.MESH,
            )

        xt_ref[:, :] = x_ref[:, :].T.astype(jnp.bfloat16)

        def cast_block(b):
            dyb_ref[:, pl.ds(b * blk, blk)] = dy_ref[
                :, pl.ds(b * blk, blk)
            ].astype(jnp.bfloat16)

        def chunk_gemm(c, b):
            return lax.dot_general(
                xt_ref[pl.ds(c * m_out, m_out), :],
                dyb_ref[:, pl.ds(b * blk, blk)],
                dimension_numbers=(((1,), (0,)), ((), ())),
                preferred_element_type=jnp.float32,
            )

        def quant(val, scale):
            return jnp.clip(
                jnp.round(val * (1.0 / scale)), -127.0, 127.0
            ).astype(jnp.int8)

        def store_local(c, b):
            p_ref[pl.ds(c * m_out, m_out), pl.ds(b * blk, blk)] = (
                chunk_gemm(c, b).astype(jnp.bfloat16)
            )

        def local(c, b):
            return p_ref[
                pl.ds(c * m_out, m_out), pl.ds(b * blk, blk)
            ].astype(jnp.float32)

        def make(slot, dest):
            return pltpu.make_async_remote_copy(
                src_ref=q_send.at[slot],
                dst_ref=recv_buf.at[slot],
                send_sem=send_sems.at[slot],
                recv_sem=recv_sems.at[slot],
                device_id=(dest,),
                device_id_type=pl.DeviceIdType.MESH,
            )

        def p1_send(c, b, slot, dest):
            q_send[slot, :, :] = quant(chunk_gemm(c, b), S1)
            s = make(slot, dest)
            s.start()
            return s

        sa_own, sa_mate, sb_own, sb_mate = [], [], [], []
        cast_block(0)
        q_send[A_OWN, :, :] = quant(chunk_gemm(q1, 0), S1)
        pl.semaphore_wait(barrier_sem, 2)
        s = make(A_OWN, q1)
        s.start()
        sa_own.append(s)
        for t in range(NS):
            base = 6 * t
            if t > 0:
                cast_block(t)
                sa_own.append(p1_send(q1, t, base + A_OWN, q1))
            cast_block(NS + t)
            sb_own.append(p1_send(q2, NS + t, base + B_OWN, q2))
            sa_mate.append(p1_send(3 - q1, t, base + A_MATE, q1))
            sb_mate.append(p1_send(q2 ^ 1, NS + t, base + B_MATE, q2))

        for t in range(NS):
            store_local(3 - my, t)
            store_local(my ^ 1, NS + t)
        for t in range(NS):
            store_local(my, t)
            store_local(my, NS + t)

        sa_p2, sb_p2 = [], []
        for t in range(NS):
            base = 6 * t
            sa_own[t].wait_recv()
            sa_mate[t].wait_recv()
            q_send[base + A_P2, :, :] = quant(
                local(3 - my, t)
                + recv_buf[base + A_MATE, :, :].astype(jnp.float32) * S1,
                S2,
            )
            s = make(base + A_P2, q2)
            s.start()
            sa_p2.append(s)
            sb_own[t].wait_recv()
            sb_mate[t].wait_recv()
            q_send[base + B_P2, :, :] = quant(
                local(my ^ 1, NS + t)
                + recv_buf[base + B_MATE, :, :].astype(jnp.float32) * S1,
                S2,
            )
            s = make(base + B_P2, q1)
            s.start()
            sb_p2.append(s)

        for t in range(NS):
            base = 6 * t
            sa_p2[t].wait_recv()
            out_ref[:, pl.ds(t * blk, blk)] = (
                local(my, t)
                + recv_buf[base + A_OWN, :, :].astype(jnp.float32) * S1
                + recv_buf[base + A_P2, :, :].astype(jnp.float32) * S2
            )
            sb_p2[t].wait_recv()
            out_ref[:, pl.ds((NS + t) * blk, blk)] = (
                local(my, NS + t)
                + recv_buf[base + B_OWN, :, :].astype(jnp.float32) * S1
                + recv_buf[base + B_P2, :, :].astype(jnp.float32) * S2
            )

        for group in (sa_own, sa_mate, sb_own, sb_mate, sa_p2, sb_p2):
            for s in group:
                s.wait_send()

    return pl.pallas_call(
        body,
        out_shape=jax.ShapeDtypeStruct((m_out, d_ff), jnp.float32),
        in_specs=[
            pl.BlockSpec(memory_space=pltpu.VMEM),
            pl.BlockSpec(memory_space=pltpu.VMEM),
        ],
        out_specs=pl.BlockSpec(memory_space=pltpu.VMEM),
        scratch_shapes=[
            pltpu.VMEM((d_model, k_per), jnp.bfloat16),
            pltpu.VMEM((k_per, d_ff), jnp.bfloat16),
            pltpu.VMEM((d_model, d_ff), jnp.bfloat16),
            pltpu.VMEM((6 * NS, m_out, blk), jnp.int8),
            pltpu.VMEM((6 * NS, m_out, blk), jnp.int8),
            pltpu.SemaphoreType.DMA((6 * NS,)),
            pltpu.SemaphoreType.DMA((6 * NS,)),
        ],
        compiler_params=pltpu.CompilerParams(collective_id=0),
    )(x, dy)
